# baseline (speedup 1.0000x reference)
import functools

import jax
import jax.numpy as jnp
from jax import lax
from jax.experimental import pallas as pl
from jax.experimental.pallas import tpu as pltpu
from jax.experimental.pallas import tpu_sc as plsc

F = 64
L = 16


def _mf_body(u_idx_hbm, i_idx_hbm, u_emb_hbm, i_emb_hbm, out_hbm,
             uidx_v, iidx_v, urows_v, irows_v, out_v, sem):
    nc = 2
    wid = lax.axis_index("s") * nc + lax.axis_index("c")
    b_per_w = out_v.shape[0]
    n_chunks = uidx_v.shape[0]
    chunk = uidx_v.shape[1]
    base = wid * b_per_w

    for j in range(n_chunks):
        pltpu.sync_copy(u_idx_hbm.at[pl.ds(base + j * chunk, chunk)],
                        uidx_v.at[j])
        pltpu.sync_copy(i_idx_hbm.at[pl.ds(base + j * chunk, chunk)],
                        iidx_v.at[j])

    copies = []
    for j in range(n_chunks):
        copies.append(pltpu.async_copy(
            u_emb_hbm.at[uidx_v.at[j]], urows_v.at[pl.ds(j * chunk, chunk)],
            sem))
        copies.append(pltpu.async_copy(
            i_emb_hbm.at[iidx_v.at[j]], irows_v.at[pl.ds(j * chunk, chunk)],
            sem))
    for c in copies:
        c.wait()

    def rb_body(rb, _):
        row0 = rb * L
        ridx = row0 + lax.broadcasted_iota(jnp.int32, (L,), 0)
        acc = jnp.zeros((L,), jnp.float32)
        for f in range(F):
            cidx = jnp.full((L,), f, jnp.int32)
            uv = plsc.load_gather(urows_v, [ridx, cidx])
            iv = plsc.load_gather(irows_v, [ridx, cidx])
            acc = acc + uv * iv
        sig = 1.0 / (1.0 + jnp.exp(-acc))
        out_v[pl.ds(row0, L)] = sig
        return 0

    lax.fori_loop(0, b_per_w // L, rb_body, 0)

    pltpu.sync_copy(out_v, out_hbm.at[pl.ds(base, b_per_w)])


def kernel(u_idx, i_idx, u_emb, i_emb):
    B = u_idx.shape[0]
    nw = 32
    b_per_w = B // nw
    chunk = 128
    n_chunks = b_per_w // chunk
    mesh = plsc.VectorSubcoreMesh(core_axis_name="c", subcore_axis_name="s")

    mf = functools.partial(
        pl.kernel, mesh=mesh,
        out_type=jax.ShapeDtypeStruct((B,), jnp.float32),
        scratch_types=[
            pltpu.VMEM((n_chunks, chunk), jnp.int32),
            pltpu.VMEM((n_chunks, chunk), jnp.int32),
            pltpu.VMEM((b_per_w, F), jnp.float32),
            pltpu.VMEM((b_per_w, F), jnp.float32),
            pltpu.VMEM((b_per_w,), jnp.float32),
            pltpu.SemaphoreType.DMA,
        ],
        compiler_params=pltpu.CompilerParams(
            needs_layout_passes=False, use_tc_tiling_on_sc=False),
    )(_mf_body)

    return mf(u_idx.astype(jnp.int32), i_idx.astype(jnp.int32), u_emb, i_emb)


# speedup vs baseline: 2.1824x; 2.1824x over previous
import functools

import jax
import jax.numpy as jnp
from jax import lax
from jax.experimental import pallas as pl
from jax.experimental.pallas import tpu as pltpu
from jax.experimental.pallas import tpu_sc as plsc

F = 64
L = 16
TILE = 8
CHUNK = 32


def _mf_body(u_idx_hbm, i_idx_hbm, u3_hbm, i3_hbm, out_hbm,
             uidx_v, iidx_v, ubuf, ibuf, out_v, sem):
    nc = 2
    wid = lax.axis_index("s") * nc + lax.axis_index("c")
    b_per_w = out_v.shape[0]
    base = wid * b_per_w

    pltpu.sync_copy(u_idx_hbm.at[pl.ds(base, b_per_w)], uidx_v)
    pltpu.sync_copy(i_idx_hbm.at[pl.ds(base, b_per_w)], iidx_v)

    lanes = lax.broadcasted_iota(jnp.int32, (L,), 0)

    def chunk_body(c, _):
        off = c * CHUNK
        uvecs, ivecs, copies = [], [], []
        for h in range(CHUNK // L):
            uvec = uidx_v[pl.ds(off + h * L, L)]
            ivec = iidx_v[pl.ds(off + h * L, L)]
            uvecs.append(uvec)
            ivecs.append(ivec)
            ut = lax.shift_right_logical(uvec, 3)
            it = lax.shift_right_logical(ivec, 3)
            for j in range(L):
                copies.append(pltpu.async_copy(
                    u3_hbm.at[pl.ds(ut[j], 1)],
                    ubuf.at[pl.ds(h * L + j, 1)], sem))
                copies.append(pltpu.async_copy(
                    i3_hbm.at[pl.ds(it[j], 1)],
                    ibuf.at[pl.ds(h * L + j, 1)], sem))
        for cp in copies:
            cp.wait()

        for h in range(CHUNK // L):
            usub = uvecs[h] & (TILE - 1)
            isub = ivecs[h] & (TILE - 1)
            res = jnp.zeros((L,), jnp.float32)
            for j in range(L):
                s = usub[j]
                t = isub[j]
                acc = jnp.zeros((L,), jnp.float32)
                for q in range(F // L):
                    uv = ubuf[h * L + j, s, pl.ds(q * L, L)]
                    iv = ibuf[h * L + j, t, pl.ds(q * L, L)]
                    acc = acc + uv * iv
                res = jnp.where(lanes == j, jnp.sum(acc), res)
            out_v[pl.ds(off + h * L, L)] = 1.0 / (1.0 + jnp.exp(-res))
        return 0

    lax.fori_loop(0, b_per_w // CHUNK, chunk_body, 0)

    pltpu.sync_copy(out_v, out_hbm.at[pl.ds(base, b_per_w)])


def kernel(u_idx, i_idx, u_emb, i_emb):
    B = u_idx.shape[0]
    nw = 32
    b_per_w = B // nw
    u3 = u_emb.reshape(u_emb.shape[0] // TILE, TILE, F)
    i3 = i_emb.reshape(i_emb.shape[0] // TILE, TILE, F)
    mesh = plsc.VectorSubcoreMesh(core_axis_name="c", subcore_axis_name="s")

    mf = functools.partial(
        pl.kernel, mesh=mesh,
        out_type=jax.ShapeDtypeStruct((B,), jnp.float32),
        scratch_types=[
            pltpu.VMEM((b_per_w,), jnp.int32),
            pltpu.VMEM((b_per_w,), jnp.int32),
            pltpu.VMEM((CHUNK, TILE, F), jnp.float32),
            pltpu.VMEM((CHUNK, TILE, F), jnp.float32),
            pltpu.VMEM((b_per_w,), jnp.float32),
            pltpu.SemaphoreType.DMA,
        ],
        compiler_params=pltpu.CompilerParams(needs_layout_passes=False),
    )(_mf_body)

    return mf(u_idx.astype(jnp.int32), i_idx.astype(jnp.int32), u3, i3)
